# baseline (speedup 1.0000x reference)
import functools

import jax
import jax.numpy as jnp
from jax import lax
from jax.experimental import pallas as pl
from jax.experimental.pallas import tpu as pltpu
from jax.experimental.pallas import tpu_sc as plsc

N_NODES = 10000
N_INC = 320000
TXT = 128
HID = 4096
F = 64

NC = 2
NS = 16
NW = NC * NS
EPT = N_INC // NW
BLK = 200
NB = EPT // BLK
N_PAD = 10240
RPT = N_PAD // NS

ROW_TILE = 400
FIN_TILE = 1000


def _xw_body(x_ref, w1t_ref, b1_ref, th_ref, xw_ref):
    f = jnp.dot(x_ref[...], w1t_ref[...], preferred_element_type=jnp.float32)
    f = jnp.maximum(f + b1_ref[...], 0.0)
    xw_ref[...] = jnp.dot(f, th_ref[...], preferred_element_type=jnp.float32)


def _xw_only(x, w1t, b1r, theta):
    n = x.shape[0]
    return pl.pallas_call(
        _xw_body,
        grid=(n // ROW_TILE,),
        in_specs=[
            pl.BlockSpec((ROW_TILE, TXT), lambda i: (i, 0)),
            pl.BlockSpec((TXT, HID), lambda i: (0, 0)),
            pl.BlockSpec((1, HID), lambda i: (0, 0)),
            pl.BlockSpec((HID, F), lambda i: (0, 0)),
        ],
        out_specs=pl.BlockSpec((ROW_TILE, F), lambda i: (i, 0)),
        out_shape=jax.ShapeDtypeStruct((N_PAD, F), jnp.float32),
    )(x, w1t, b1r, theta)


def _feat_body(x_ref, w1t_ref, b1_ref, feat_ref):
    f = jnp.dot(x_ref[...], w1t_ref[...], preferred_element_type=jnp.float32)
    feat_ref[...] = jnp.maximum(f + b1_ref[...], 0.0)


def _feat_only(x, w1t, b1r):
    n = x.shape[0]
    return pl.pallas_call(
        _feat_body,
        grid=(n // ROW_TILE,),
        in_specs=[
            pl.BlockSpec((ROW_TILE, TXT), lambda i: (i, 0)),
            pl.BlockSpec((TXT, HID), lambda i: (0, 0)),
            pl.BlockSpec((1, HID), lambda i: (0, 0)),
        ],
        out_specs=pl.BlockSpec((ROW_TILE, HID), lambda i: (i, 0)),
        out_shape=jax.ShapeDtypeStruct((n, HID), jnp.float32),
    )(x, w1t, b1r)


def _seg_main(table_s, acc_s, cnt_s, sidx_v, didx_v, rows, ones_v,
              gsem, ssem, csem):
    gd = [None, None]
    sd = [None, None]
    cds = []
    gd[0] = pltpu.async_copy(table_s.at[sidx_v.at[0]], rows[0], gsem[0])
    for b in range(NB):
        cur = b & 1
        nxt = cur ^ 1
        if b + 1 < NB:
            if b >= 1:
                sd[nxt].wait()
            gd[nxt] = pltpu.async_copy(table_s.at[sidx_v.at[b + 1]], rows[nxt], gsem[nxt])
        gd[cur].wait()
        cds.append(pltpu.async_copy(ones_v, cnt_s.at[didx_v.at[b]], csem, add=True))
        sd[cur] = pltpu.async_copy(rows[cur], acc_s.at[didx_v.at[b]], ssem[cur], add=True)
    sd[(NB - 1) & 1].wait()
    if NB > 1:
        sd[NB & 1].wait()
    for d in cds:
        d.wait()


def _seg_epilogue(acc_s, cnt_s, acc_part, cnt_part, c, row0):
    pltpu.sync_copy(acc_s.at[pl.ds(row0, RPT)], acc_part.at[c, pl.ds(row0, RPT)])
    pltpu.sync_copy(cnt_s.at[pl.ds(row0, RPT)], cnt_part.at[c, pl.ds(row0, RPT)])


def _seg1_body(table, sidx3, didx3, z2, z1, ones_h,
               acc_part, cnt_part,
               table_s, acc_s, cnt_s, sidx_v, didx_v, rows0, rows1, ones_v,
               gsem0, gsem1, ssem0, ssem1, csem):
    c = lax.axis_index("c")
    s = lax.axis_index("s")
    wid = c * NS + s
    row0 = pl.multiple_of(s * RPT, 8)

    pltpu.sync_copy(table.at[pl.ds(row0, RPT)], table_s.at[pl.ds(row0, RPT)])
    pltpu.sync_copy(z2, acc_s.at[pl.ds(row0, RPT)])
    pltpu.sync_copy(z1, cnt_s.at[pl.ds(row0, RPT)])
    pltpu.sync_copy(ones_h, ones_v)
    pltpu.sync_copy(sidx3.at[wid], sidx_v)
    pltpu.sync_copy(didx3.at[wid], didx_v)
    plsc.subcore_barrier()
    _seg_main(table_s, acc_s, cnt_s, sidx_v, didx_v, (rows0, rows1), ones_v,
              (gsem0, gsem1), (ssem0, ssem1), csem)
    plsc.subcore_barrier()
    _seg_epilogue(acc_s, cnt_s, acc_part, cnt_part, c, row0)


_SEG_OUT = (
    jax.ShapeDtypeStruct((NC, N_PAD, F), jnp.float32),
    jax.ShapeDtypeStruct((NC, N_PAD), jnp.float32),
)
_SEG_MESH = plsc.VectorSubcoreMesh(
    core_axis_name="c", subcore_axis_name="s", num_cores=NC, num_subcores=NS
)
_SEG_SCRATCH = [
    pltpu.VMEM_SHARED((N_PAD, F), jnp.float32),
    pltpu.VMEM_SHARED((N_PAD, F), jnp.float32),
    pltpu.VMEM_SHARED((N_PAD,), jnp.float32),
    pltpu.VMEM((NB, BLK), jnp.int32),
    pltpu.VMEM((NB, BLK), jnp.int32),
    pltpu.VMEM((BLK, F), jnp.float32),
    pltpu.VMEM((BLK, F), jnp.float32),
    pltpu.VMEM((BLK,), jnp.float32),
]
_SEG_SEMS = [pltpu.SemaphoreType.DMA] * 5

_seg1_call = pl.kernel(
    _seg1_body,
    out_type=_SEG_OUT,
    mesh=_SEG_MESH,
    scratch_types=_SEG_SCRATCH + _SEG_SEMS,
    compiler_params=pltpu.CompilerParams(use_tc_tiling_on_sc=False),
)


def _comb_body(acc_ref, cnt_ref, out_ref):
    a = acc_ref[0] + acc_ref[1]
    cnt = cnt_ref[0] + cnt_ref[1]
    inv = jnp.where(cnt > 0.0, 1.0 / cnt, 0.0)
    out_ref[...] = a * inv


def _combine(acc_p, cnt_p3):
    return pl.pallas_call(
        _comb_body,
        grid=(N_NODES // FIN_TILE,),
        in_specs=[
            pl.BlockSpec((NC, FIN_TILE, F), lambda i: (0, i, 0)),
            pl.BlockSpec((NC, FIN_TILE, 1), lambda i: (0, i, 0)),
        ],
        out_specs=pl.BlockSpec((FIN_TILE, F), lambda i: (i, 0)),
        out_shape=jax.ShapeDtypeStruct((N_PAD, F), jnp.float32),
    )(acc_p, cnt_p3)


def _fin_body(acc_ref, cnt_ref, bias_ref, hid_ref, code_ref):
    a = acc_ref[0] + acc_ref[1]
    cnt = cnt_ref[0] + cnt_ref[1]
    inv = jnp.where(cnt > 0.0, 1.0 / cnt, 0.0)
    h = a * inv + bias_ref[...]
    hid_ref[...] = h
    code_ref[...] = jnp.tanh(h)


def _final(acc_p, cnt_p3, biasr):
    n = N_NODES
    return pl.pallas_call(
        _fin_body,
        grid=(n // FIN_TILE,),
        in_specs=[
            pl.BlockSpec((NC, FIN_TILE, F), lambda i: (0, i, 0)),
            pl.BlockSpec((NC, FIN_TILE, 1), lambda i: (0, i, 0)),
            pl.BlockSpec((1, F), lambda i: (0, 0)),
        ],
        out_specs=[
            pl.BlockSpec((FIN_TILE, F), lambda i: (i, 0)),
            pl.BlockSpec((FIN_TILE, F), lambda i: (i, 0)),
        ],
        out_shape=[
            jax.ShapeDtypeStruct((n, F), jnp.float32),
            jax.ShapeDtypeStruct((n, F), jnp.float32),
        ],
    )(acc_p, cnt_p3, biasr)


def kernel(x, G, W1, b1, theta, bias_h):
    w1t = W1.T
    b1r = b1.reshape(1, HID)
    biasr = bias_h.reshape(1, F)
    nidx = G[0]
    hidx = G[1]
    nidx3 = nidx.reshape(NW, NB, BLK)
    hidx3 = hidx.reshape(NW, NB, BLK)

    xw = _xw_only(x, w1t, b1r, theta)

    z2 = jnp.zeros((RPT, F), jnp.float32)
    z1 = jnp.zeros((RPT,), jnp.float32)
    ones = jnp.ones((BLK,), jnp.float32)

    he_acc_p, bcnt_p = _seg1_call(xw, nidx3, hidx3, z2, z1, ones)
    feat = _feat_only(x, w1t, b1r)
    he_feat = _combine(he_acc_p, bcnt_p.reshape(NC, N_PAD, 1))
    nd_acc_p, dcnt_p = _seg1_call(he_feat, hidx3, nidx3, z2, z1, ones)
    hid, code = _final(nd_acc_p, dcnt_p.reshape(NC, N_PAD, 1), biasr)
    return (feat, hid, code)


# speedup vs baseline: 1.2017x; 1.2017x over previous
import functools

import jax
import jax.numpy as jnp
from jax import lax
from jax.experimental import pallas as pl
from jax.experimental.pallas import tpu as pltpu
from jax.experimental.pallas import tpu_sc as plsc

N_NODES = 10000
N_INC = 320000
TXT = 128
HID = 4096
F = 64

NC = 2
NS = 16
NW = NC * NS
EPT = N_INC // NW
BLK = 500
NB = EPT // BLK
N_PAD = 10240
RPT = N_PAD // NS

ROW_TILE = 400
FIN_TILE = 1000


def _mm_body(x_ref, w1t_ref, b1_ref, th_ref, feat_ref, xw_ref):
    f = jnp.dot(x_ref[...], w1t_ref[...], preferred_element_type=jnp.float32)
    f = jnp.maximum(f + b1_ref[...], 0.0)
    feat_ref[...] = f
    xw_ref[...] = jnp.dot(f, th_ref[...], preferred_element_type=jnp.float32)


def _matmuls(x, w1t, b1r, theta):
    n = x.shape[0]
    return pl.pallas_call(
        _mm_body,
        grid=(n // ROW_TILE,),
        in_specs=[
            pl.BlockSpec((ROW_TILE, TXT), lambda i: (i, 0)),
            pl.BlockSpec((TXT, HID), lambda i: (0, 0)),
            pl.BlockSpec((1, HID), lambda i: (0, 0)),
            pl.BlockSpec((HID, F), lambda i: (0, 0)),
        ],
        out_specs=[
            pl.BlockSpec((ROW_TILE, HID), lambda i: (i, 0)),
            pl.BlockSpec((ROW_TILE, F), lambda i: (i, 0)),
        ],
        out_shape=[
            jax.ShapeDtypeStruct((n, HID), jnp.float32),
            jax.ShapeDtypeStruct((n, F), jnp.float32),
        ],
    )(x, w1t, b1r, theta)


def _seg_main(table_s, acc_s, cnt_s, sidx_v, didx_v, rows, ones_v,
              gsem, ssem, csem):
    gd = [None, None]
    sd = [None, None]
    cds = []
    gd[0] = pltpu.async_copy(table_s.at[sidx_v.at[0]], rows[0], gsem[0])
    for b in range(NB):
        cur = b & 1
        nxt = cur ^ 1
        if b + 1 < NB:
            if b >= 1:
                sd[nxt].wait()
            gd[nxt] = pltpu.async_copy(table_s.at[sidx_v.at[b + 1]], rows[nxt], gsem[nxt])
        gd[cur].wait()
        cds.append(pltpu.async_copy(ones_v, cnt_s.at[didx_v.at[b]], csem, add=True))
        sd[cur] = pltpu.async_copy(rows[cur], acc_s.at[didx_v.at[b]], ssem[cur], add=True)
    sd[(NB - 1) & 1].wait()
    if NB > 1:
        sd[NB & 1].wait()
    for d in cds:
        d.wait()


def _seg_epilogue(acc_s, cnt_s, acc_part, cnt_part, c, row0):
    pltpu.sync_copy(acc_s.at[pl.ds(row0, RPT)], acc_part.at[c, pl.ds(row0, RPT)])
    pltpu.sync_copy(cnt_s.at[pl.ds(row0, RPT)], cnt_part.at[c, pl.ds(row0, RPT)])


def _seg1_body(table, sidx3, didx3, z2, z1, ones_h,
               acc_part, cnt_part,
               acc_s, cnt_s, sidx_v, didx_v, rows0, rows1, ones_v,
               gsem0, gsem1, ssem0, ssem1, csem):
    c = lax.axis_index("c")
    s = lax.axis_index("s")
    wid = c * NS + s
    row0 = pl.multiple_of(s * RPT, 8)

    pltpu.sync_copy(z2, acc_s.at[pl.ds(row0, RPT)])
    pltpu.sync_copy(z1, cnt_s.at[pl.ds(row0, RPT)])
    pltpu.sync_copy(ones_h, ones_v)
    pltpu.sync_copy(sidx3.at[wid], sidx_v)
    pltpu.sync_copy(didx3.at[wid], didx_v)
    plsc.subcore_barrier()
    _seg_main(table, acc_s, cnt_s, sidx_v, didx_v, (rows0, rows1), ones_v,
              (gsem0, gsem1), (ssem0, ssem1), csem)
    plsc.subcore_barrier()
    _seg_epilogue(acc_s, cnt_s, acc_part, cnt_part, c, row0)


_SEG_OUT = (
    jax.ShapeDtypeStruct((NC, N_PAD, F), jnp.float32),
    jax.ShapeDtypeStruct((NC, N_PAD), jnp.float32),
)
_SEG_MESH = plsc.VectorSubcoreMesh(
    core_axis_name="c", subcore_axis_name="s", num_cores=NC, num_subcores=NS
)
_SEG_SCRATCH = [
    pltpu.VMEM_SHARED((N_PAD, F), jnp.float32),
    pltpu.VMEM_SHARED((N_PAD,), jnp.float32),
    pltpu.VMEM((NB, BLK), jnp.int32),
    pltpu.VMEM((NB, BLK), jnp.int32),
    pltpu.VMEM((BLK, F), jnp.float32),
    pltpu.VMEM((BLK, F), jnp.float32),
    pltpu.VMEM((BLK,), jnp.float32),
]
_SEG_SEMS = [pltpu.SemaphoreType.DMA] * 5

_seg1_call = pl.kernel(
    _seg1_body,
    out_type=_SEG_OUT,
    mesh=_SEG_MESH,
    scratch_types=_SEG_SCRATCH + _SEG_SEMS,
    compiler_params=pltpu.CompilerParams(use_tc_tiling_on_sc=False),
)


def _comb_body(acc_ref, cnt_ref, out_ref):
    a = acc_ref[0] + acc_ref[1]
    cnt = cnt_ref[0] + cnt_ref[1]
    inv = jnp.where(cnt > 0.0, 1.0 / cnt, 0.0)
    out_ref[...] = a * inv


def _combine(acc_p, cnt_p3):
    return pl.pallas_call(
        _comb_body,
        grid=(N_NODES // FIN_TILE,),
        in_specs=[
            pl.BlockSpec((NC, FIN_TILE, F), lambda i: (0, i, 0)),
            pl.BlockSpec((NC, FIN_TILE, 1), lambda i: (0, i, 0)),
        ],
        out_specs=pl.BlockSpec((FIN_TILE, F), lambda i: (i, 0)),
        out_shape=jax.ShapeDtypeStruct((N_PAD, F), jnp.float32),
    )(acc_p, cnt_p3)


def _fin_body(acc_ref, cnt_ref, bias_ref, hid_ref, code_ref):
    a = acc_ref[0] + acc_ref[1]
    cnt = cnt_ref[0] + cnt_ref[1]
    inv = jnp.where(cnt > 0.0, 1.0 / cnt, 0.0)
    h = a * inv + bias_ref[...]
    hid_ref[...] = h
    code_ref[...] = jnp.tanh(h)


def _final(acc_p, cnt_p3, biasr):
    n = N_NODES
    return pl.pallas_call(
        _fin_body,
        grid=(n // FIN_TILE,),
        in_specs=[
            pl.BlockSpec((NC, FIN_TILE, F), lambda i: (0, i, 0)),
            pl.BlockSpec((NC, FIN_TILE, 1), lambda i: (0, i, 0)),
            pl.BlockSpec((1, F), lambda i: (0, 0)),
        ],
        out_specs=[
            pl.BlockSpec((FIN_TILE, F), lambda i: (i, 0)),
            pl.BlockSpec((FIN_TILE, F), lambda i: (i, 0)),
        ],
        out_shape=[
            jax.ShapeDtypeStruct((n, F), jnp.float32),
            jax.ShapeDtypeStruct((n, F), jnp.float32),
        ],
    )(acc_p, cnt_p3, biasr)


def kernel(x, G, W1, b1, theta, bias_h):
    w1t = W1.T
    b1r = b1.reshape(1, HID)
    biasr = bias_h.reshape(1, F)
    nidx = G[0]
    hidx = G[1]
    nidx3 = nidx.reshape(NW, NB, BLK)
    hidx3 = hidx.reshape(NW, NB, BLK)

    feat, xw = _matmuls(x, w1t, b1r, theta)

    z2 = jnp.zeros((RPT, F), jnp.float32)
    z1 = jnp.zeros((RPT,), jnp.float32)
    ones = jnp.ones((BLK,), jnp.float32)

    he_acc_p, bcnt_p = _seg1_call(xw, nidx3, hidx3, z2, z1, ones)
    he_feat = _combine(he_acc_p, bcnt_p.reshape(NC, N_PAD, 1))
    nd_acc_p, dcnt_p = _seg1_call(he_feat, hidx3, nidx3, z2, z1, ones)
    hid, code = _final(nd_acc_p, dcnt_p.reshape(NC, N_PAD, 1), biasr)
    return (feat, hid, code)


# speedup vs baseline: 1.3376x; 1.1131x over previous
import functools

import jax
import jax.numpy as jnp
from jax import lax
from jax.experimental import pallas as pl
from jax.experimental.pallas import tpu as pltpu
from jax.experimental.pallas import tpu_sc as plsc

N_NODES = 10000
N_INC = 320000
TXT = 128
HID = 4096
F = 64

NC = 2
NS = 16
NW = NC * NS
EPT = N_INC // NW
BLK = 500
NB = EPT // BLK
N_PAD = 10240
RPT = N_PAD // NS

ROW_TILE = 400
FIN_TILE = 1280


def _mm_body(x_ref, w1_ref, b1_ref, th_ref, feat_ref, xw_ref):
    f = lax.dot_general(x_ref[...], w1_ref[...], (((1,), (1,)), ((), ())),
                        preferred_element_type=jnp.float32)
    f = jnp.maximum(f + b1_ref[...], 0.0)
    feat_ref[...] = f
    xw_ref[...] = jnp.dot(f, th_ref[...], preferred_element_type=jnp.float32)


def _matmuls(x, w1, b1r, theta):
    n = x.shape[0]
    return pl.pallas_call(
        _mm_body,
        grid=(n // ROW_TILE,),
        in_specs=[
            pl.BlockSpec((ROW_TILE, TXT), lambda i: (i, 0)),
            pl.BlockSpec((HID, TXT), lambda i: (0, 0)),
            pl.BlockSpec((1, HID), lambda i: (0, 0)),
            pl.BlockSpec((HID, F), lambda i: (0, 0)),
        ],
        out_specs=[
            pl.BlockSpec((ROW_TILE, HID), lambda i: (i, 0)),
            pl.BlockSpec((ROW_TILE, F), lambda i: (i, 0)),
        ],
        out_shape=[
            jax.ShapeDtypeStruct((n, HID), jnp.float32),
            jax.ShapeDtypeStruct((n, F), jnp.float32),
        ],
    )(x, w1, b1r, theta)


def _seg_main(table_s, acc_s, cnt_s, sidx_v, didx_v, rows, ones_v,
              gsem, ssem, csem):
    gd = [None, None]
    sd = [None, None]
    cds = []
    gd[0] = pltpu.async_copy(table_s.at[sidx_v.at[0]], rows[0], gsem[0])
    for b in range(NB):
        cur = b & 1
        nxt = cur ^ 1
        if b + 1 < NB:
            if b >= 1:
                sd[nxt].wait()
            gd[nxt] = pltpu.async_copy(table_s.at[sidx_v.at[b + 1]], rows[nxt], gsem[nxt])
        gd[cur].wait()
        cds.append(pltpu.async_copy(ones_v, cnt_s.at[didx_v.at[b]], csem, add=True))
        sd[cur] = pltpu.async_copy(rows[cur], acc_s.at[didx_v.at[b]], ssem[cur], add=True)
    sd[(NB - 1) & 1].wait()
    if NB > 1:
        sd[NB & 1].wait()
    for d in cds:
        d.wait()


def _seg_epilogue(acc_s, cnt_s, acc_part, cnt_part, c, row0):
    pltpu.sync_copy(acc_s.at[pl.ds(row0, RPT)], acc_part.at[c, pl.ds(row0, RPT)])
    pltpu.sync_copy(cnt_s.at[pl.ds(row0, RPT)], cnt_part.at[c, pl.ds(row0, RPT)])


def _make_seg_body(sr, dr):
    def body(table, g4, z2, z1, ones_h,
             acc_part, cnt_part,
             acc_s, cnt_s, sidx_v, didx_v, rows0, rows1, ones_v,
             gsem0, gsem1, ssem0, ssem1, csem):
        c = lax.axis_index("c")
        s = lax.axis_index("s")
        wid = c * NS + s
        row0 = pl.multiple_of(s * RPT, 8)

        pltpu.sync_copy(z2, acc_s.at[pl.ds(row0, RPT)])
        pltpu.sync_copy(z1, cnt_s.at[pl.ds(row0, RPT)])
        pltpu.sync_copy(ones_h, ones_v)
        pltpu.sync_copy(g4.at[sr, wid], sidx_v)
        pltpu.sync_copy(g4.at[dr, wid], didx_v)
        plsc.subcore_barrier()
        _seg_main(table, acc_s, cnt_s, sidx_v, didx_v, (rows0, rows1), ones_v,
                  (gsem0, gsem1), (ssem0, ssem1), csem)
        plsc.subcore_barrier()
        _seg_epilogue(acc_s, cnt_s, acc_part, cnt_part, c, row0)
    return body


_SEG_OUT = (
    jax.ShapeDtypeStruct((NC, N_PAD, F), jnp.float32),
    jax.ShapeDtypeStruct((NC, N_PAD), jnp.float32),
)
_SEG_MESH = plsc.VectorSubcoreMesh(
    core_axis_name="c", subcore_axis_name="s", num_cores=NC, num_subcores=NS
)
_SEG_SCRATCH = [
    pltpu.VMEM_SHARED((N_PAD, F), jnp.float32),
    pltpu.VMEM_SHARED((N_PAD,), jnp.float32),
    pltpu.VMEM((NB, BLK), jnp.int32),
    pltpu.VMEM((NB, BLK), jnp.int32),
    pltpu.VMEM((BLK, F), jnp.float32),
    pltpu.VMEM((BLK, F), jnp.float32),
    pltpu.VMEM((BLK,), jnp.float32),
]
_SEG_SEMS = [pltpu.SemaphoreType.DMA] * 5

_seg_ps1 = pl.kernel(
    _make_seg_body(0, 1),
    out_type=_SEG_OUT,
    mesh=_SEG_MESH,
    scratch_types=_SEG_SCRATCH + _SEG_SEMS,
    compiler_params=pltpu.CompilerParams(use_tc_tiling_on_sc=False),
)

_seg_ps2 = pl.kernel(
    _make_seg_body(1, 0),
    out_type=_SEG_OUT,
    mesh=_SEG_MESH,
    scratch_types=_SEG_SCRATCH + _SEG_SEMS,
    compiler_params=pltpu.CompilerParams(use_tc_tiling_on_sc=False),
)


def _comb_body(acc_ref, cnt_ref, out_ref):
    a = acc_ref[0] + acc_ref[1]
    cnt = cnt_ref[0] + cnt_ref[1]
    inv = jnp.where(cnt > 0.0, 1.0 / cnt, 0.0)
    out_ref[...] = a * inv[:, None]


def _combine(acc_p, cnt_p):
    return pl.pallas_call(
        _comb_body,
        grid=(N_PAD // FIN_TILE,),
        in_specs=[
            pl.BlockSpec((NC, FIN_TILE, F), lambda i: (0, i, 0)),
            pl.BlockSpec((NC, FIN_TILE), lambda i: (0, i)),
        ],
        out_specs=pl.BlockSpec((FIN_TILE, F), lambda i: (i, 0)),
        out_shape=jax.ShapeDtypeStruct((N_PAD, F), jnp.float32),
    )(acc_p, cnt_p)


def _fin_body(acc_ref, cnt_ref, bias_ref, hid_ref, code_ref):
    a = acc_ref[0] + acc_ref[1]
    cnt = cnt_ref[0] + cnt_ref[1]
    inv = jnp.where(cnt > 0.0, 1.0 / cnt, 0.0)
    h = a * inv[:, None] + bias_ref[...]
    hid_ref[...] = h
    code_ref[...] = jnp.tanh(h)


def _final(acc_p, cnt_p, biasr):
    n = N_NODES
    return pl.pallas_call(
        _fin_body,
        grid=(N_PAD // FIN_TILE,),
        in_specs=[
            pl.BlockSpec((NC, FIN_TILE, F), lambda i: (0, i, 0)),
            pl.BlockSpec((NC, FIN_TILE), lambda i: (0, i)),
            pl.BlockSpec((1, F), lambda i: (0, 0)),
        ],
        out_specs=[
            pl.BlockSpec((FIN_TILE, F), lambda i: (i, 0)),
            pl.BlockSpec((FIN_TILE, F), lambda i: (i, 0)),
        ],
        out_shape=[
            jax.ShapeDtypeStruct((n, F), jnp.float32),
            jax.ShapeDtypeStruct((n, F), jnp.float32),
        ],
    )(acc_p, cnt_p, biasr)


def kernel(x, G, W1, b1, theta, bias_h):
    b1r = b1.reshape(1, HID)
    biasr = bias_h.reshape(1, F)
    g4 = G.reshape(2, NW, NB, BLK)

    feat, xw = _matmuls(x, W1, b1r, theta)

    z2 = jnp.zeros((RPT, F), jnp.float32)
    z1 = jnp.zeros((RPT,), jnp.float32)
    ones = jnp.ones((BLK,), jnp.float32)

    he_acc_p, bcnt_p = _seg_ps1(xw, g4, z2, z1, ones)
    he_feat = _combine(he_acc_p, bcnt_p)
    nd_acc_p, dcnt_p = _seg_ps2(he_feat, g4, z2, z1, ones)
    hid, code = _final(nd_acc_p, dcnt_p, biasr)
    return (feat, hid, code)


# speedup vs baseline: 1.3406x; 1.0022x over previous
import functools

import jax
import jax.numpy as jnp
from jax import lax
from jax.experimental import pallas as pl
from jax.experimental.pallas import tpu as pltpu
from jax.experimental.pallas import tpu_sc as plsc

N_NODES = 10000
N_INC = 320000
TXT = 128
HID = 4096
F = 64

NC = 2
NS = 16
NW = NC * NS
EPT = N_INC // NW
BLK = 500
NB = EPT // BLK
N_PAD = 10240
RPT = N_PAD // NS

ROW_TILE = 400
FIN_TILE = 1280


def _mm_body(x_ref, w1_ref, b1_ref, th_ref, feat_ref, xw_ref):
    f = lax.dot_general(x_ref[...], w1_ref[...], (((1,), (1,)), ((), ())),
                        preferred_element_type=jnp.float32)
    f = jnp.maximum(f + b1_ref[...], 0.0)
    feat_ref[...] = f
    xw_ref[...] = jnp.dot(f, th_ref[...], preferred_element_type=jnp.float32)


def _matmuls(x, w1, b1r, theta):
    n = x.shape[0]
    return pl.pallas_call(
        _mm_body,
        grid=(n // ROW_TILE,),
        in_specs=[
            pl.BlockSpec((ROW_TILE, TXT), lambda i: (i, 0)),
            pl.BlockSpec((HID, TXT), lambda i: (0, 0)),
            pl.BlockSpec((1, HID), lambda i: (0, 0)),
            pl.BlockSpec((HID, F), lambda i: (0, 0)),
        ],
        out_specs=[
            pl.BlockSpec((ROW_TILE, HID), lambda i: (i, 0)),
            pl.BlockSpec((ROW_TILE, F), lambda i: (i, 0)),
        ],
        out_shape=[
            jax.ShapeDtypeStruct((n, HID), jnp.float32),
            jax.ShapeDtypeStruct((n, F), jnp.float32),
        ],
    )(x, w1, b1r, theta)


def _seg_main(table_s, acc_s, cnt_s, sidx_v, didx_v, rows, ones_v,
              gsem, ssem, csem):
    gd = [None, None]
    sd = [None, None]
    cd = [None, None]
    gd[0] = pltpu.async_copy(table_s.at[sidx_v.at[0]], rows[0], gsem[0])
    for b in range(NB):
        cur = b & 1
        nxt = cur ^ 1
        if b + 1 < NB:
            if b >= 1:
                sd[nxt].wait()
                cd[nxt].wait()
            gd[nxt] = pltpu.async_copy(table_s.at[sidx_v.at[b + 1]], rows[nxt], gsem[nxt])
        gd[cur].wait()
        cd[cur] = pltpu.async_copy(ones_v, cnt_s.at[didx_v.at[b]], csem, add=True)
        sd[cur] = pltpu.async_copy(rows[cur], acc_s.at[didx_v.at[b]], ssem[cur], add=True)
    sd[(NB - 1) & 1].wait()
    cd[(NB - 1) & 1].wait()
    if NB > 1:
        sd[NB & 1].wait()
        cd[NB & 1].wait()


def _seg_epilogue(acc_s, cnt_s, acc_part, cnt_part, c, row0):
    pltpu.sync_copy(acc_s.at[pl.ds(row0, RPT)], acc_part.at[c, pl.ds(row0, RPT)])
    pltpu.sync_copy(cnt_s.at[pl.ds(row0, RPT)], cnt_part.at[c, pl.ds(row0, RPT)])


def _make_seg_body(sr, dr):
    def body(table, g4, z2, z1, ones_h,
             acc_part, cnt_part,
             acc_s, cnt_s, sidx_v, didx_v, rows0, rows1, ones_v,
             gsem0, gsem1, ssem0, ssem1, csem):
        c = lax.axis_index("c")
        s = lax.axis_index("s")
        wid = c * NS + s
        row0 = pl.multiple_of(s * RPT, 8)

        pltpu.sync_copy(z2, acc_s.at[pl.ds(row0, RPT)])
        pltpu.sync_copy(z1, cnt_s.at[pl.ds(row0, RPT)])
        pltpu.sync_copy(ones_h, ones_v)
        pltpu.sync_copy(g4.at[sr, wid], sidx_v)
        pltpu.sync_copy(g4.at[dr, wid], didx_v)
        plsc.subcore_barrier()
        _seg_main(table, acc_s, cnt_s, sidx_v, didx_v, (rows0, rows1), ones_v,
                  (gsem0, gsem1), (ssem0, ssem1), csem)
        plsc.subcore_barrier()
        _seg_epilogue(acc_s, cnt_s, acc_part, cnt_part, c, row0)
    return body


_SEG_OUT = (
    jax.ShapeDtypeStruct((NC, N_PAD, F), jnp.float32),
    jax.ShapeDtypeStruct((NC, N_PAD), jnp.float32),
)
_SEG_MESH = plsc.VectorSubcoreMesh(
    core_axis_name="c", subcore_axis_name="s", num_cores=NC, num_subcores=NS
)
_SEG_SCRATCH = [
    pltpu.VMEM_SHARED((N_PAD, F), jnp.float32),
    pltpu.VMEM_SHARED((N_PAD,), jnp.float32),
    pltpu.VMEM((NB, BLK), jnp.int32),
    pltpu.VMEM((NB, BLK), jnp.int32),
    pltpu.VMEM((BLK, F), jnp.float32),
    pltpu.VMEM((BLK, F), jnp.float32),
    pltpu.VMEM((BLK,), jnp.float32),
]
_SEG_SEMS = [pltpu.SemaphoreType.DMA] * 5

_seg_ps1 = pl.kernel(
    _make_seg_body(0, 1),
    out_type=_SEG_OUT,
    mesh=_SEG_MESH,
    scratch_types=_SEG_SCRATCH + _SEG_SEMS,
    compiler_params=pltpu.CompilerParams(use_tc_tiling_on_sc=False),
)

_seg_ps2 = pl.kernel(
    _make_seg_body(1, 0),
    out_type=_SEG_OUT,
    mesh=_SEG_MESH,
    scratch_types=_SEG_SCRATCH + _SEG_SEMS,
    compiler_params=pltpu.CompilerParams(use_tc_tiling_on_sc=False),
)


def _comb_body(acc_ref, cnt_ref, out_ref):
    a = acc_ref[0] + acc_ref[1]
    cnt = cnt_ref[0] + cnt_ref[1]
    inv = jnp.where(cnt > 0.0, 1.0 / cnt, 0.0)
    out_ref[...] = a * inv[:, None]


def _combine(acc_p, cnt_p):
    return pl.pallas_call(
        _comb_body,
        grid=(N_PAD // FIN_TILE,),
        in_specs=[
            pl.BlockSpec((NC, FIN_TILE, F), lambda i: (0, i, 0)),
            pl.BlockSpec((NC, FIN_TILE), lambda i: (0, i)),
        ],
        out_specs=pl.BlockSpec((FIN_TILE, F), lambda i: (i, 0)),
        out_shape=jax.ShapeDtypeStruct((N_PAD, F), jnp.float32),
    )(acc_p, cnt_p)


def _fin_body(acc_ref, cnt_ref, bias_ref, hid_ref, code_ref):
    a = acc_ref[0] + acc_ref[1]
    cnt = cnt_ref[0] + cnt_ref[1]
    inv = jnp.where(cnt > 0.0, 1.0 / cnt, 0.0)
    h = a * inv[:, None] + bias_ref[...]
    hid_ref[...] = h
    code_ref[...] = jnp.tanh(h)


def _final(acc_p, cnt_p, biasr):
    n = N_NODES
    return pl.pallas_call(
        _fin_body,
        grid=(N_PAD // FIN_TILE,),
        in_specs=[
            pl.BlockSpec((NC, FIN_TILE, F), lambda i: (0, i, 0)),
            pl.BlockSpec((NC, FIN_TILE), lambda i: (0, i)),
            pl.BlockSpec((1, F), lambda i: (0, 0)),
        ],
        out_specs=[
            pl.BlockSpec((FIN_TILE, F), lambda i: (i, 0)),
            pl.BlockSpec((FIN_TILE, F), lambda i: (i, 0)),
        ],
        out_shape=[
            jax.ShapeDtypeStruct((n, F), jnp.float32),
            jax.ShapeDtypeStruct((n, F), jnp.float32),
        ],
    )(acc_p, cnt_p, biasr)


def kernel(x, G, W1, b1, theta, bias_h):
    b1r = b1.reshape(1, HID)
    biasr = bias_h.reshape(1, F)
    g4 = G.reshape(2, NW, NB, BLK)

    feat, xw = _matmuls(x, W1, b1r, theta)

    z2 = jnp.zeros((RPT, F), jnp.float32)
    z1 = jnp.zeros((RPT,), jnp.float32)
    ones = jnp.ones((BLK,), jnp.float32)

    he_acc_p, bcnt_p = _seg_ps1(xw, g4, z2, z1, ones)
    he_feat = _combine(he_acc_p, bcnt_p)
    nd_acc_p, dcnt_p = _seg_ps2(he_feat, g4, z2, z1, ones)
    hid, code = _final(nd_acc_p, dcnt_p, biasr)
    return (feat, hid, code)
